# baseline (speedup 1.0000x reference)
import functools

import jax
import jax.numpy as jnp
from jax import lax
from jax.experimental import pallas as pl
from jax.experimental.pallas import tpu as pltpu
from jax.experimental.pallas import tpu_sc as plsc

_SEQ = 200
_BATCH = 4096
_DIM = 64
_NCORES = 2
_NSUB = 16
_NW = _NCORES * _NSUB
_BPW = _BATCH // _NW
_CH0 = 128
_CH1 = _SEQ - _CH0
_PACK_C = 65536

_SC_PARAMS = pltpu.CompilerParams(
    use_tc_tiling_on_sc=False, needs_layout_passes=False)


def _prep_indices(texts):
    mesh = plsc.VectorSubcoreMesh(core_axis_name="c", subcore_axis_name="s")

    @functools.partial(
        pl.kernel,
        out_type=jax.ShapeDtypeStruct((_BATCH, _SEQ), jnp.int32),
        mesh=mesh,
        scratch_types=[
            pltpu.VMEM((_SEQ, _BPW), jnp.int32),
            pltpu.VMEM((_BPW, _SEQ), jnp.int32),
        ],
        compiler_params=_SC_PARAMS,
    )
    def kern(texts_hbm, out_hbm, idx_sv, idx_v):
        wid = lax.axis_index("c") * _NSUB + lax.axis_index("s")
        base = wid * _BPW
        pltpu.sync_copy(texts_hbm.at[:, pl.ds(base, _BPW)], idx_sv)

        @pl.loop(0, _BPW)
        def _(b):
            colv = jnp.full((16,), b, jnp.int32)
            for k in range(13):
                s0 = 184 if k == 12 else 16 * k
                rows = lax.iota(jnp.int32, 16) + s0
                v = plsc.load_gather(idx_sv, [rows, colv])
                low = (v & jnp.int32(_PACK_C // 4 - 1)) << jnp.int32(2)
                q = (v >> jnp.int32(_PACK_C.bit_length() - 3)) & jnp.int32(3)
                idx_v[b, pl.ds(s0, 16)] = (
                    (v & jnp.int32(~(_PACK_C - 1))) + low + q)

        pltpu.sync_copy(idx_v, out_hbm.at[pl.ds(base, _BPW)])

    return kern(texts)


def _cbow_pool(idx_t, emb):
    mesh = plsc.VectorSubcoreMesh(core_axis_name="c", subcore_axis_name="s")

    @functools.partial(
        pl.kernel,
        out_type=jax.ShapeDtypeStruct((_BATCH, _DIM), jnp.float32),
        mesh=mesh,
        scratch_types=[
            pltpu.VMEM((_BPW, _SEQ), jnp.int32),
            pltpu.VMEM((_SEQ, _DIM // 2), jnp.int32),
            pltpu.VMEM((_SEQ, _DIM // 2), jnp.int32),
            pltpu.VMEM((_SEQ, _DIM // 2), jnp.int32),
            pltpu.VMEM((_SEQ, _DIM // 2), jnp.int32),
            pltpu.VMEM((_BPW, _DIM), jnp.float32),
            pltpu.SemaphoreType.DMA,
            pltpu.SemaphoreType.DMA,
            pltpu.SemaphoreType.DMA,
            pltpu.SemaphoreType.DMA,
        ],
        compiler_params=_SC_PARAMS,
    )
    def kern(idx_hbm, emb_hbm, out_hbm, idx_v, bufa, bufb, bufc, bufd,
             out_v, sema, semb, semc, semd):
        wid = lax.axis_index("c") * _NSUB + lax.axis_index("s")
        base = wid * _BPW
        pltpu.sync_copy(idx_hbm.at[pl.ds(base, _BPW)], idx_v)

        def copies(b, buf, sem):
            return (
                pltpu.make_async_copy(
                    emb_hbm.at[idx_v.at[b, pl.ds(0, _CH0)]],
                    buf.at[pl.ds(0, _CH0)], sem),
                pltpu.make_async_copy(
                    emb_hbm.at[idx_v.at[b, pl.ds(_CH0, _CH1)]],
                    buf.at[pl.ds(_CH0, _CH1)], sem),
            )

        def fire(b, buf, sem):
            for c in copies(b, buf, sem):
                c.start()

        def drain(b, buf, sem):
            for c in copies(b, buf, sem):
                c.wait()

        def accum(b, buf):
            sh = jnp.int32(16)

            def expand_lo(w):
                return plsc.bitcast(w << sh, jnp.float32)

            def expand_hi(w):
                return plsc.bitcast(w, jnp.float32)

            def body(i, acc):
                s = i * 2
                w0 = buf[s, pl.ds(0, 16)]
                w1 = buf[s, pl.ds(16, 16)]
                w2 = buf[s + 1, pl.ds(0, 16)]
                w3 = buf[s + 1, pl.ds(16, 16)]
                return (
                    acc[0] + expand_lo(w0),
                    acc[1] + expand_lo(w1),
                    acc[2] + expand_hi(w0),
                    acc[3] + expand_hi(w1),
                    acc[4] + expand_lo(w2),
                    acc[5] + expand_lo(w3),
                    acc[6] + expand_hi(w2),
                    acc[7] + expand_hi(w3),
                )

            z = jnp.zeros((16,), jnp.float32)
            a = lax.fori_loop(0, _SEQ // 2, body, (z,) * 8, unroll=4)
            inv = jnp.float32(1.0 / _SEQ)
            for c in range(4):
                out_v[b, pl.ds(16 * c, 16)] = (a[c] + a[c + 4]) * inv

        ring = ((bufa, sema), (bufb, semb), (bufc, semc), (bufd, semd))
        for r in range(3):
            fire(r, *ring[r])

        @pl.loop(0, _BPW, step=4)
        def _(b):
            for r in range(4):
                nxt = b + r + 3

                @pl.when(nxt < _BPW)
                def _(nxt=nxt, r=r):
                    fire(nxt, *ring[(r + 3) % 4])

                drain(b + r, *ring[r])
                accum(b + r, ring[r][0])

        pltpu.sync_copy(out_v, out_hbm.at[pl.ds(base, _BPW)])

    return kern(idx_t, emb)


def _pack_table(emb_t, vocab):
    nblk = pl.cdiv(vocab, _PACK_C)
    q = _PACK_C // 4

    def pack_words(x):
        bits = jax.lax.bitcast_convert_type(x, jnp.int32)
        rnd = (bits + jnp.int32(0x7FFF) + ((bits >> 16) & 1)) >> 16
        return (rnd[0:32, :] & jnp.int32(0xFFFF)) | (rnd[32:64, :] << 16)

    def body(x_ref, o_ref):
        for i in range(q // 128):
            w4 = jnp.concatenate(
                [pack_words(x_ref[:, q * t + 128 * i:q * t + 128 * (i + 1)])
                 for t in range(4)], axis=0)
            o_ref[128 * i:128 * (i + 1), :] = w4.T

    return pl.pallas_call(
        body,
        grid=(nblk,),
        in_specs=[pl.BlockSpec((_DIM, _PACK_C), lambda j: (0, j))],
        out_specs=pl.BlockSpec((q, 2 * _DIM), lambda j: (j, 0)),
        out_shape=jax.ShapeDtypeStruct((nblk * q, 2 * _DIM), jnp.int32),
    )(emb_t)


def _mlp_head(cbow, W1, b1, W2, b2):
    def body(x_ref, w1_ref, b1_ref, w2_ref, b2_ref, o_ref):
        x = x_ref[...]
        h = jnp.maximum(
            jnp.dot(x, w1_ref[...], preferred_element_type=jnp.float32)
            + b1_ref[...], 0.0)
        o_ref[...] = (
            jnp.dot(h, w2_ref[...], preferred_element_type=jnp.float32)
            + b2_ref[...])

    return pl.pallas_call(
        body,
        out_shape=jax.ShapeDtypeStruct((_BATCH, b2.shape[-1]), jnp.float32),
    )(cbow, W1, b1.reshape(1, -1), W2, b2.reshape(1, -1))


def kernel(texts, emb, W1, b1, W2, b2):
    vocab = emb.shape[0]
    idx_t = _prep_indices(texts.astype(jnp.int32))
    packed = _pack_table(emb.T, vocab)
    embl = packed.reshape(4 * packed.shape[0], _DIM // 2)
    cbow = _cbow_pool(idx_t, embl)
    return _mlp_head(cbow, W1, b1, W2, b2)


# speedup vs baseline: 1.0486x; 1.0486x over previous
import functools

import jax
import jax.numpy as jnp
from jax import lax
from jax.experimental import pallas as pl
from jax.experimental.pallas import tpu as pltpu
from jax.experimental.pallas import tpu_sc as plsc

_SEQ = 200
_BATCH = 4096
_DIM = 64
_NCORES = 2
_NSUB = 16
_NW = _NCORES * _NSUB
_BPW = _BATCH // _NW
_CH0 = 128
_CH1 = _SEQ - _CH0
_PACK_C = 32768

_SC_PARAMS = pltpu.CompilerParams(
    use_tc_tiling_on_sc=False, needs_layout_passes=False)


def _prep_indices(texts):
    mesh = plsc.VectorSubcoreMesh(core_axis_name="c", subcore_axis_name="s")

    @functools.partial(
        pl.kernel,
        out_type=jax.ShapeDtypeStruct((_BATCH, _SEQ), jnp.int32),
        mesh=mesh,
        scratch_types=[
            pltpu.VMEM((_SEQ, _BPW), jnp.int32),
            pltpu.VMEM((_BPW, _SEQ), jnp.int32),
        ],
        compiler_params=_SC_PARAMS,
    )
    def kern(texts_hbm, out_hbm, idx_sv, idx_v):
        wid = lax.axis_index("c") * _NSUB + lax.axis_index("s")
        base = wid * _BPW
        pltpu.sync_copy(texts_hbm.at[:, pl.ds(base, _BPW)], idx_sv)

        @pl.loop(0, _BPW)
        def _(b):
            colv = jnp.full((16,), b, jnp.int32)
            for k in range(13):
                s0 = 184 if k == 12 else 16 * k
                rows = lax.iota(jnp.int32, 16) + s0
                v = plsc.load_gather(idx_sv, [rows, colv])
                low = (v & jnp.int32(_PACK_C // 4 - 1)) << jnp.int32(2)
                q = (v >> jnp.int32(_PACK_C.bit_length() - 3)) & jnp.int32(3)
                idx_v[b, pl.ds(s0, 16)] = (
                    (v & jnp.int32(~(_PACK_C - 1))) + low + q)

        pltpu.sync_copy(idx_v, out_hbm.at[pl.ds(base, _BPW)])

    return kern(texts)


def _cbow_pool(idx_t, emb):
    mesh = plsc.VectorSubcoreMesh(core_axis_name="c", subcore_axis_name="s")

    @functools.partial(
        pl.kernel,
        out_type=jax.ShapeDtypeStruct((_BATCH, _DIM), jnp.float32),
        mesh=mesh,
        scratch_types=[
            pltpu.VMEM((_BPW, _SEQ), jnp.int32),
            pltpu.VMEM((_SEQ, _DIM // 2), jnp.int32),
            pltpu.VMEM((_SEQ, _DIM // 2), jnp.int32),
            pltpu.VMEM((_SEQ, _DIM // 2), jnp.int32),
            pltpu.VMEM((_SEQ, _DIM // 2), jnp.int32),
            pltpu.VMEM((_SEQ, _DIM // 2), jnp.int32),
            pltpu.VMEM((_SEQ, _DIM // 2), jnp.int32),
            pltpu.VMEM((_SEQ, _DIM // 2), jnp.int32),
            pltpu.VMEM((_SEQ, _DIM // 2), jnp.int32),
            pltpu.VMEM((_BPW, _DIM), jnp.float32),
            pltpu.SemaphoreType.DMA,
            pltpu.SemaphoreType.DMA,
            pltpu.SemaphoreType.DMA,
            pltpu.SemaphoreType.DMA,
            pltpu.SemaphoreType.DMA,
            pltpu.SemaphoreType.DMA,
            pltpu.SemaphoreType.DMA,
            pltpu.SemaphoreType.DMA,
        ],
        compiler_params=_SC_PARAMS,
    )
    def kern(idx_hbm, emb_hbm, out_hbm, idx_v, bufa, bufb, bufc, bufd,
             bufe, buff, bufg, bufh, out_v,
             sema, semb, semc, semd, seme, semf, semg, semh):
        wid = lax.axis_index("c") * _NSUB + lax.axis_index("s")
        base = wid * _BPW
        pltpu.sync_copy(idx_hbm.at[pl.ds(base, _BPW)], idx_v)

        def copies(b, buf, sem):
            return (
                pltpu.make_async_copy(
                    emb_hbm.at[idx_v.at[b, pl.ds(0, _CH0)]],
                    buf.at[pl.ds(0, _CH0)], sem),
                pltpu.make_async_copy(
                    emb_hbm.at[idx_v.at[b, pl.ds(_CH0, _CH1)]],
                    buf.at[pl.ds(_CH0, _CH1)], sem),
            )

        def fire(b, buf, sem):
            for c in copies(b, buf, sem):
                c.start()

        def drain(b, buf, sem):
            for c in copies(b, buf, sem):
                c.wait()

        def accum(b, buf):
            sh = jnp.int32(16)

            def expand_lo(w):
                return plsc.bitcast(w << sh, jnp.float32)

            def expand_hi(w):
                return plsc.bitcast(w, jnp.float32)

            def body(i, acc):
                s = i * 2
                w0 = buf[s, pl.ds(0, 16)]
                w1 = buf[s, pl.ds(16, 16)]
                w2 = buf[s + 1, pl.ds(0, 16)]
                w3 = buf[s + 1, pl.ds(16, 16)]
                return (
                    acc[0] + expand_lo(w0),
                    acc[1] + expand_lo(w1),
                    acc[2] + expand_hi(w0),
                    acc[3] + expand_hi(w1),
                    acc[4] + expand_lo(w2),
                    acc[5] + expand_lo(w3),
                    acc[6] + expand_hi(w2),
                    acc[7] + expand_hi(w3),
                )

            z = jnp.zeros((16,), jnp.float32)
            a = lax.fori_loop(0, _SEQ // 2, body, (z,) * 8, unroll=4)
            inv = jnp.float32(1.0 / _SEQ)
            for c in range(4):
                out_v[b, pl.ds(16 * c, 16)] = (a[c] + a[c + 4]) * inv

        ring = ((bufa, sema), (bufb, semb), (bufc, semc), (bufd, semd),
                (bufe, seme), (buff, semf), (bufg, semg), (bufh, semh))
        nring = len(ring)
        for r in range(nring - 1):
            fire(r, *ring[r])

        @pl.loop(0, _BPW, step=nring)
        def _(b):
            for r in range(nring):
                nxt = b + r + nring - 1

                @pl.when(nxt < _BPW)
                def _(nxt=nxt, r=r):
                    fire(nxt, *ring[(r + nring - 1) % nring])

                drain(b + r, *ring[r])
                accum(b + r, ring[r][0])

        pltpu.sync_copy(out_v, out_hbm.at[pl.ds(base, _BPW)])

    return kern(idx_t, emb)


def _pack_table(emb_t, vocab):
    nblk = pl.cdiv(vocab, _PACK_C)
    q = _PACK_C // 4

    def pack_words(x):
        bits = jax.lax.bitcast_convert_type(x, jnp.int32)
        rnd = (bits + jnp.int32(0x7FFF) + ((bits >> 16) & 1)) >> 16
        return (rnd[0:32, :] & jnp.int32(0xFFFF)) | (rnd[32:64, :] << 16)

    def body(x_ref, o_ref):
        for i in range(q // 128):
            w4 = jnp.concatenate(
                [pack_words(x_ref[:, q * t + 128 * i:q * t + 128 * (i + 1)])
                 for t in range(4)], axis=0)
            o_ref[128 * i:128 * (i + 1), :] = w4.T

    return pl.pallas_call(
        body,
        grid=(nblk,),
        in_specs=[pl.BlockSpec((_DIM, _PACK_C), lambda j: (0, j))],
        out_specs=pl.BlockSpec((q, 2 * _DIM), lambda j: (j, 0)),
        out_shape=jax.ShapeDtypeStruct((nblk * q, 2 * _DIM), jnp.int32),
    )(emb_t)


def _mlp_head(cbow, W1, b1, W2, b2):
    def body(x_ref, w1_ref, b1_ref, w2_ref, b2_ref, o_ref):
        x = x_ref[...]
        h = jnp.maximum(
            jnp.dot(x, w1_ref[...], preferred_element_type=jnp.float32)
            + b1_ref[...], 0.0)
        o_ref[...] = (
            jnp.dot(h, w2_ref[...], preferred_element_type=jnp.float32)
            + b2_ref[...])

    return pl.pallas_call(
        body,
        out_shape=jax.ShapeDtypeStruct((_BATCH, b2.shape[-1]), jnp.float32),
    )(cbow, W1, b1.reshape(1, -1), W2, b2.reshape(1, -1))


def kernel(texts, emb, W1, b1, W2, b2):
    vocab = emb.shape[0]
    idx_t = _prep_indices(texts.astype(jnp.int32))
    packed = _pack_table(emb.T, vocab)
    embl = packed.reshape(4 * packed.shape[0], _DIM // 2)
    cbow = _cbow_pool(idx_t, embl)
    return _mlp_head(cbow, W1, b1, W2, b2)
